# baseline (speedup 1.0000x reference)
import functools

import jax
import jax.numpy as jnp
from jax import lax
from jax.experimental import pallas as pl
from jax.experimental.pallas import tpu as pltpu
from jax.experimental.pallas import tpu_sc as plsc

_B = 16384
_VOCAB = 1000000
_L = 50
_EMB = 64
_PROJ = 128
_NW = 32
_BPW = _B // _NW
_NG = _BPW // 128

_sc_mesh = plsc.VectorSubcoreMesh(core_axis_name="c", subcore_axis_name="s",
                                  num_cores=2, num_subcores=16)


_CW = 256
_NCH = (_VOCAB - 64) // _CW
_SPAN = (_NCH + _NW - 1) // _NW
_ND = 3


@functools.partial(
    pl.kernel,
    out_type=jax.ShapeDtypeStruct((_VOCAB * _EMB,), jnp.float32),
    mesh=_sc_mesh,
    scratch_types=[
        pltpu.VMEM((_EMB, _CW), jnp.float32),
        pltpu.VMEM((_EMB, _CW), jnp.float32),
        pltpu.VMEM((_EMB, _CW), jnp.float32),
        pltpu.VMEM((_EMB * _CW,), jnp.float32),
        pltpu.VMEM((_EMB * _CW,), jnp.float32),
        pltpu.VMEM((_EMB * _CW,), jnp.float32),
        pltpu.SemaphoreType.DMA,
        pltpu.SemaphoreType.DMA,
        pltpu.SemaphoreType.DMA,
        pltpu.SemaphoreType.DMA,
        pltpu.SemaphoreType.DMA,
        pltpu.SemaphoreType.DMA,
    ],
    compiler_params=pltpu.CompilerParams(use_tc_tiling_on_sc=True,
                                         needs_layout_passes=False),
)
def _sc_relayout(embt_hbm, out_hbm, in0, in1, in2, ot0, ot1, ot2,
                 si0, si1, si2, so0, so1, so2):
    w = lax.axis_index("s") * 2 + lax.axis_index("c")
    start = w * _SPAN
    nvalid = jnp.minimum(_SPAN, _NCH - start)

    iota = lax.iota(jnp.int32, 16)
    col_pats = [iota + 16 * a for a in range(_CW // 16)]

    def _in_start(c, buf, sem):
        pltpu.async_copy(embt_hbm.at[:, pl.ds(c * _CW, _CW)], buf, sem)

    def _in_wait(c, buf, sem):
        pltpu.make_async_copy(embt_hbm.at[:, pl.ds(c * _CW, _CW)], buf,
                              sem).wait()

    def _out_start(c, buf, sem):
        pltpu.async_copy(buf, out_hbm.at[pl.ds(c * _CW * _EMB, _CW * _EMB)],
                         sem)

    def _out_wait(c, buf, sem):
        pltpu.make_async_copy(buf,
                              out_hbm.at[pl.ds(c * _CW * _EMB, _CW * _EMB)],
                              sem).wait()

    def _transpose(src, dst):
        @plsc.parallel_loop(0, 16, 1, unroll=2)
        def _tb(d):
            m = (iota + d) & 15
            s_d = iota * 64 + m
            rows = [m + 16 * b for b in range(4)]
            for a in range(_CW // 16):
                for b in range(4):
                    vec = plsc.load_gather(src, [rows[b], col_pats[a]])
                    plsc.store_scatter(dst, [s_d + (1024 * a + 16 * b)],
                                       vec)

    bufs = ((in0, ot0, si0, so0), (in1, ot1, si1, so1),
            (in2, ot2, si2, so2))

    for q in range(_ND):
        @pl.when(nvalid > q)
        def _(q=q):
            _in_start(start + q, bufs[q][0], bufs[q][2])

    def _slot(s, cbuf, obuf, semi, semo):
        c = start + s

        @pl.when(s < nvalid)
        def _():
            _in_wait(c, cbuf, semi)

            @pl.when(s >= _ND)
            def _():
                _out_wait(c - _ND, obuf, semo)

            _transpose(cbuf, obuf)
            _out_start(c, obuf, semo)

            @pl.when(s + _ND < nvalid)
            def _():
                _in_start(c + _ND, cbuf, semi)

    def _body(t, carry):
        for q in range(_ND):
            _slot(_ND * t + q, bufs[q][0], bufs[q][1], bufs[q][2],
                  bufs[q][3])
        return carry

    lax.fori_loop(0, (_SPAN + _ND - 1) // _ND, _body, 0)

    for q in range(_ND):
        @pl.when(nvalid > q)
        def _(q=q):
            s_q = ((nvalid - 1 - q) // _ND) * _ND + q
            _out_wait(start + s_q, bufs[q][1], bufs[q][3])


@functools.partial(
    pl.kernel,
    out_type=(
        jax.ShapeDtypeStruct((_B, _EMB), jnp.float32),
        jax.ShapeDtypeStruct((_B, _EMB), jnp.float32),
    ),
    mesh=_sc_mesh,
    scratch_types=[
        pltpu.VMEM((_L, _NG, 128), jnp.int32),
        pltpu.VMEM((_BPW, _EMB), jnp.float32),
        pltpu.VMEM((_BPW, _EMB), jnp.float32),
        pltpu.SemaphoreType.DMA,
        pltpu.SemaphoreType.DMA,
    ],
    compiler_params=pltpu.CompilerParams(use_tc_tiling_on_sc=False),
)
def _sc_pool(idx_hbm, emb_hbm, out0_hbm, out1_hbm, idx_v,
             acc0, acc1, sem0, sem1):
    w = lax.axis_index("s") * 2 + lax.axis_index("c")
    base = w * _BPW
    pltpu.sync_copy(idx_hbm.at[:, pl.ds(w * _NG, _NG), :], idx_v)

    def _start(j, acc, sem, add):
        for c in range(_NG):
            pltpu.async_copy(
                emb_hbm.at[idx_v.at[j, c]],
                acc.at[pl.ds(c * 128, 128), :],
                sem,
                add=add,
            )

    def _wait(j, acc, sem):
        for c in range(_NG):
            pltpu.make_async_copy(
                emb_hbm.at[idx_v.at[j, c]],
                acc.at[pl.ds(c * 128, 128), :],
                sem,
            ).wait()

    _start(0, acc0, sem0, False)
    _start(1, acc1, sem1, False)

    def _body(i, carry):
        je = 2 * i + 2
        jo = 2 * i + 3
        _wait(je - 2, acc0, sem0)
        _start(je, acc0, sem0, True)
        _wait(jo - 2, acc1, sem1)
        _start(jo, acc1, sem1, True)
        return carry

    lax.fori_loop(0, (_L - 2) // 2, _body, 0)

    _wait(_L - 2, acc0, sem0)
    _wait(_L - 1, acc1, sem1)

    pltpu.sync_copy(acc0, out0_hbm.at[pl.ds(base, _BPW), :])
    pltpu.sync_copy(acc1, out1_hbm.at[pl.ds(base, _BPW), :])


_BLK = 2048


def _tc_body(idx_ref, s0_ref, s1_ref, emb0_ref, W1_ref, b1_ref,
             W2_ref, b2_ref, rec_ref, proj_ref):
    cnt = jnp.sum((idx_ref[...] != 0).astype(jnp.float32), axis=1,
                  keepdims=True)
    s = s0_ref[...] + s1_ref[...]
    ms = s - (_L - cnt) * emb0_ref[...]
    rec = jnp.where(cnt > 0.0, ms / (cnt + 1e-8), 0.0)
    nrm = jnp.sqrt(jnp.sum(rec * rec, axis=1, keepdims=True))
    rec = rec / jnp.maximum(nrm, 1e-12)
    rec_ref[...] = rec
    h = lax.dot_general(rec, W1_ref[...], (((1,), (1,)), ((), ())),
                        preferred_element_type=jnp.float32) + b1_ref[...]
    h = jnp.maximum(h, 0.0)
    p = lax.dot_general(h, W2_ref[...], (((1,), (1,)), ((), ())),
                        preferred_element_type=jnp.float32) + b2_ref[...]
    pn = jnp.sqrt(jnp.sum(p * p, axis=1, keepdims=True))
    proj_ref[...] = p / jnp.maximum(pn, 1e-12)


_tc_post = pl.pallas_call(
    _tc_body,
    grid=(_B // _BLK,),
    in_specs=[
        pl.BlockSpec((_BLK, _L), lambda i: (i, 0)),
        pl.BlockSpec((_BLK, _EMB), lambda i: (i, 0)),
        pl.BlockSpec((_BLK, _EMB), lambda i: (i, 0)),
        pl.BlockSpec((1, _EMB), lambda i: (0, 0)),
        pl.BlockSpec((_EMB, _EMB), lambda i: (0, 0)),
        pl.BlockSpec((1, _EMB), lambda i: (0, 0)),
        pl.BlockSpec((_PROJ, _EMB), lambda i: (0, 0)),
        pl.BlockSpec((1, _PROJ), lambda i: (0, 0)),
    ],
    out_specs=[
        pl.BlockSpec((_BLK, _EMB), lambda i: (i, 0)),
        pl.BlockSpec((_BLK, _PROJ), lambda i: (i, 0)),
    ],
    out_shape=[
        jax.ShapeDtypeStruct((_B, _EMB), jnp.float32),
        jax.ShapeDtypeStruct((_B, _PROJ), jnp.float32),
    ],
)


@jax.jit
def kernel(indices, emb, W1, b1, W2, b2):
    idx = indices.astype(jnp.int32)
    idx_t = idx.T.reshape(_L, _B // 128, 128)
    emb_rows = _sc_relayout(emb.T).reshape(_VOCAB // 2, 2 * _EMB)
    tail = emb[_NCH * _CW:].reshape(32, 2 * _EMB)
    emb_rows = lax.dynamic_update_slice(emb_rows, tail, (_NCH * _CW // 2, 0))
    emb_lin = emb_rows.reshape(_VOCAB, _EMB)
    s0, s1 = _sc_pool(idx_t, emb_lin)
    rec, proj = _tc_post(idx, s0, s1, emb[0:1], W1,
                         b1.reshape(1, _EMB), W2, b2.reshape(1, _PROJ))
    return rec, proj


# speedup vs baseline: 1.1287x; 1.1287x over previous
import functools

import jax
import jax.numpy as jnp
from jax import lax
from jax.experimental import pallas as pl
from jax.experimental.pallas import tpu as pltpu
from jax.experimental.pallas import tpu_sc as plsc

_B = 16384
_VOCAB = 1000000
_L = 50
_EMB = 64
_PROJ = 128
_NW = 32
_BPW = _B // _NW
_NG = _BPW // 128

_sc_mesh = plsc.VectorSubcoreMesh(core_axis_name="c", subcore_axis_name="s",
                                  num_cores=2, num_subcores=16)


_CW = 128
_NCH = (_VOCAB - 64) // _CW
_SPAN = (_NCH + _NW - 1) // _NW
_ND = 4


@functools.partial(
    pl.kernel,
    out_type=jax.ShapeDtypeStruct((_VOCAB * _EMB,), jnp.float32),
    mesh=_sc_mesh,
    scratch_types=[
        pltpu.VMEM((_EMB, _CW), jnp.float32),
        pltpu.VMEM((_EMB, _CW), jnp.float32),
        pltpu.VMEM((_EMB, _CW), jnp.float32),
        pltpu.VMEM((_EMB, _CW), jnp.float32),
        pltpu.VMEM((_EMB * _CW,), jnp.float32),
        pltpu.VMEM((_EMB * _CW,), jnp.float32),
        pltpu.VMEM((_EMB * _CW,), jnp.float32),
        pltpu.VMEM((_EMB * _CW,), jnp.float32),
        pltpu.SemaphoreType.DMA,
        pltpu.SemaphoreType.DMA,
        pltpu.SemaphoreType.DMA,
        pltpu.SemaphoreType.DMA,
        pltpu.SemaphoreType.DMA,
        pltpu.SemaphoreType.DMA,
        pltpu.SemaphoreType.DMA,
        pltpu.SemaphoreType.DMA,
    ],
    compiler_params=pltpu.CompilerParams(use_tc_tiling_on_sc=True,
                                         needs_layout_passes=False),
)
def _sc_relayout(embt_hbm, out_hbm, in0, in1, in2, in3, ot0, ot1, ot2, ot3,
                 si0, si1, si2, si3, so0, so1, so2, so3):
    w = lax.axis_index("s") * 2 + lax.axis_index("c")
    start = w * _SPAN
    nvalid = jnp.minimum(_SPAN, _NCH - start)

    iota = lax.iota(jnp.int32, 16)
    col_pats = [iota + 16 * a for a in range(_CW // 16)]

    def _in_start(c, buf, sem):
        pltpu.async_copy(embt_hbm.at[:, pl.ds(c * _CW, _CW)], buf, sem)

    def _in_wait(c, buf, sem):
        pltpu.make_async_copy(embt_hbm.at[:, pl.ds(c * _CW, _CW)], buf,
                              sem).wait()

    def _out_start(c, buf, sem):
        pltpu.async_copy(buf, out_hbm.at[pl.ds(c * _CW * _EMB, _CW * _EMB)],
                         sem)

    def _out_wait(c, buf, sem):
        pltpu.make_async_copy(buf,
                              out_hbm.at[pl.ds(c * _CW * _EMB, _CW * _EMB)],
                              sem).wait()

    def _transpose(src, dst):
        @plsc.parallel_loop(0, 16, 1, unroll=2)
        def _tb(d):
            m = (iota + d) & 15
            s_d = iota * 64 + m
            rows = [m + 16 * b for b in range(4)]
            for a in range(_CW // 16):
                for b in range(4):
                    vec = plsc.load_gather(src, [rows[b], col_pats[a]])
                    plsc.store_scatter(dst, [s_d + (1024 * a + 16 * b)],
                                       vec)

    bufs = ((in0, ot0, si0, so0), (in1, ot1, si1, so1),
            (in2, ot2, si2, so2), (in3, ot3, si3, so3))

    for q in range(_ND):
        @pl.when(nvalid > q)
        def _(q=q):
            _in_start(start + q, bufs[q][0], bufs[q][2])

    def _slot(s, cbuf, obuf, semi, semo):
        c = start + s

        @pl.when(s < nvalid)
        def _():
            _in_wait(c, cbuf, semi)

            @pl.when(s >= _ND)
            def _():
                _out_wait(c - _ND, obuf, semo)

            _transpose(cbuf, obuf)
            _out_start(c, obuf, semo)

            @pl.when(s + _ND < nvalid)
            def _():
                _in_start(c + _ND, cbuf, semi)

    def _body(t, carry):
        for q in range(_ND):
            _slot(_ND * t + q, bufs[q][0], bufs[q][1], bufs[q][2],
                  bufs[q][3])
        return carry

    lax.fori_loop(0, (_SPAN + _ND - 1) // _ND, _body, 0)

    for q in range(_ND):
        @pl.when(nvalid > q)
        def _(q=q):
            s_q = ((nvalid - 1 - q) // _ND) * _ND + q
            _out_wait(start + s_q, bufs[q][1], bufs[q][3])


@functools.partial(
    pl.kernel,
    out_type=(
        jax.ShapeDtypeStruct((_B, _EMB), jnp.float32),
        jax.ShapeDtypeStruct((_B, _EMB), jnp.float32),
    ),
    mesh=_sc_mesh,
    scratch_types=[
        pltpu.VMEM((_L, _NG, 128), jnp.int32),
        pltpu.VMEM((_BPW, _EMB), jnp.float32),
        pltpu.VMEM((_BPW, _EMB), jnp.float32),
        pltpu.SemaphoreType.DMA,
        pltpu.SemaphoreType.DMA,
    ],
    compiler_params=pltpu.CompilerParams(use_tc_tiling_on_sc=False),
)
def _sc_pool(idx_hbm, emb_hbm, out0_hbm, out1_hbm, idx_v,
             acc0, acc1, sem0, sem1):
    w = lax.axis_index("s") * 2 + lax.axis_index("c")
    base = w * _BPW
    pltpu.sync_copy(idx_hbm.at[:, pl.ds(w * _NG, _NG), :], idx_v)

    def _start(j, acc, sem, add):
        for c in range(_NG):
            pltpu.async_copy(
                emb_hbm.at[idx_v.at[j, c]],
                acc.at[pl.ds(c * 128, 128), :],
                sem,
                add=add,
            )

    def _wait(j, acc, sem):
        for c in range(_NG):
            pltpu.make_async_copy(
                emb_hbm.at[idx_v.at[j, c]],
                acc.at[pl.ds(c * 128, 128), :],
                sem,
            ).wait()

    _start(0, acc0, sem0, False)
    _start(1, acc1, sem1, False)

    def _body(i, carry):
        je = 2 * i + 2
        jo = 2 * i + 3
        _wait(je - 2, acc0, sem0)
        _start(je, acc0, sem0, True)
        _wait(jo - 2, acc1, sem1)
        _start(jo, acc1, sem1, True)
        return carry

    lax.fori_loop(0, (_L - 2) // 2, _body, 0)

    _wait(_L - 2, acc0, sem0)
    _wait(_L - 1, acc1, sem1)

    pltpu.sync_copy(acc0, out0_hbm.at[pl.ds(base, _BPW), :])
    pltpu.sync_copy(acc1, out1_hbm.at[pl.ds(base, _BPW), :])


_BLK = 2048


def _tc_body(idx_ref, s0_ref, s1_ref, emb0_ref, W1_ref, b1_ref,
             W2_ref, b2_ref, rec_ref, proj_ref):
    cnt = jnp.sum((idx_ref[...] != 0).astype(jnp.float32), axis=1,
                  keepdims=True)
    s = s0_ref[...] + s1_ref[...]
    ms = s - (_L - cnt) * emb0_ref[...]
    rec = jnp.where(cnt > 0.0, ms / (cnt + 1e-8), 0.0)
    nrm = jnp.sqrt(jnp.sum(rec * rec, axis=1, keepdims=True))
    rec = rec / jnp.maximum(nrm, 1e-12)
    rec_ref[...] = rec
    h = lax.dot_general(rec, W1_ref[...], (((1,), (1,)), ((), ())),
                        preferred_element_type=jnp.float32) + b1_ref[...]
    h = jnp.maximum(h, 0.0)
    p = lax.dot_general(h, W2_ref[...], (((1,), (1,)), ((), ())),
                        preferred_element_type=jnp.float32) + b2_ref[...]
    pn = jnp.sqrt(jnp.sum(p * p, axis=1, keepdims=True))
    proj_ref[...] = p / jnp.maximum(pn, 1e-12)


_tc_post = pl.pallas_call(
    _tc_body,
    grid=(_B // _BLK,),
    in_specs=[
        pl.BlockSpec((_BLK, _L), lambda i: (i, 0)),
        pl.BlockSpec((_BLK, _EMB), lambda i: (i, 0)),
        pl.BlockSpec((_BLK, _EMB), lambda i: (i, 0)),
        pl.BlockSpec((1, _EMB), lambda i: (0, 0)),
        pl.BlockSpec((_EMB, _EMB), lambda i: (0, 0)),
        pl.BlockSpec((1, _EMB), lambda i: (0, 0)),
        pl.BlockSpec((_PROJ, _EMB), lambda i: (0, 0)),
        pl.BlockSpec((1, _PROJ), lambda i: (0, 0)),
    ],
    out_specs=[
        pl.BlockSpec((_BLK, _EMB), lambda i: (i, 0)),
        pl.BlockSpec((_BLK, _PROJ), lambda i: (i, 0)),
    ],
    out_shape=[
        jax.ShapeDtypeStruct((_B, _EMB), jnp.float32),
        jax.ShapeDtypeStruct((_B, _PROJ), jnp.float32),
    ],
)


@jax.jit
def kernel(indices, emb, W1, b1, W2, b2):
    idx = indices.astype(jnp.int32)
    idx_t = idx.T.reshape(_L, _B // 128, 128)
    emb_rows = _sc_relayout(emb.T).reshape(_VOCAB // 2, 2 * _EMB)
    tail = emb[_NCH * _CW:].reshape(32, 2 * _EMB)
    emb_rows = lax.dynamic_update_slice(emb_rows, tail, (_NCH * _CW // 2, 0))
    emb_lin = emb_rows.reshape(_VOCAB, _EMB)
    s0, s1 = _sc_pool(idx_t, emb_lin)
    rec, proj = _tc_post(idx, s0, s1, emb[0:1], W1,
                         b1.reshape(1, _EMB), W2, b2.reshape(1, _PROJ))
    return rec, proj
